# baseline (speedup 1.0000x reference)
import functools

import jax
import jax.numpy as jnp
from jax import lax
from jax.experimental import pallas as pl
from jax.experimental.pallas import tpu as pltpu
from jax.experimental.pallas import tpu_sc as plsc

VOCAB = 1000
D = 2048
F = 8
B, S = 2, 2048
N = B * S
NSC = 2816
PT = N - NSC
PB = 256
VP = 1024

NC, NS, L = 2, 16, 16
NW = NC * NS
P_W = NSC // NW
C = 2
G = C * F
NCHUNK = P_W // C

_mesh = plsc.VectorSubcoreMesh(core_axis_name="c", subcore_axis_name="s")


@functools.partial(
    pl.kernel,
    mesh=_mesh,
    out_type=jax.ShapeDtypeStruct((NSC, D), jnp.float32),
    scratch_types=[
        pltpu.VMEM((NCHUNK, G), jnp.int32),
        pltpu.VMEM((G, D), jnp.float32),
        pltpu.VMEM((G, D), jnp.float32),
        pltpu.VMEM((C, D), jnp.float32),
        pltpu.VMEM((C, D), jnp.float32),
        pltpu.SemaphoreType.DMA,
        pltpu.SemaphoreType.DMA,
        pltpu.SemaphoreType.DMA,
        pltpu.SemaphoreType.DMA,
    ],
)
def _embed_sc(x_hbm, w_hbm, out_hbm, idx_v, rows0, rows1,
              acc0, acc1, sem0, sem1, wsem0, wsem1):
    wid = lax.axis_index("s") * NC + lax.axis_index("c")
    row_base = wid * P_W
    bufs = (rows0, rows1)
    sems = (sem0, sem1)
    accs = (acc0, acc1)
    wsems = (wsem0, wsem1)

    pltpu.sync_copy(x_hbm.at[wid], idx_v)
    offs = (lax.iota(jnp.int32, 16) % F) * VOCAB

    def _bias(c, carry):
        idx_v[c, pl.ds(0, L)] = idx_v[c, pl.ds(0, L)] + offs
        return carry

    lax.fori_loop(0, NCHUNK, _bias, 0)

    pltpu.async_copy(w_hbm.at[idx_v.at[0]], rows0, sem0)

    def _pair(k2, carry):
        for b in range(2):
            k = k2 * 2 + b
            nb = 1 - b

            @pl.when(k + 1 < NCHUNK)
            def _():
                pltpu.async_copy(w_hbm.at[idx_v.at[k + 1]], bufs[nb], sems[nb])

            pltpu.make_async_copy(w_hbm.at[idx_v.at[k]], bufs[b], sems[b]).wait()
            rows_v = bufs[b]
            acc_v = accs[b]

            @pl.when(k >= 2)
            def _():
                pltpu.make_async_copy(
                    acc_v, out_hbm.at[pl.ds(row_base + (k - 2) * C, C)], wsems[b]
                ).wait()

            @plsc.parallel_loop(0, D // L, unroll=8)
            def _cols(j):
                col = j * L
                for c in range(C):
                    t = [rows_v[c * F + f, pl.ds(col, L)] for f in range(F)]
                    while len(t) > 1:
                        t = [a + b2 for a, b2 in zip(t[::2], t[1::2])]
                    acc_v[c, pl.ds(col, L)] = t[0]

            pltpu.async_copy(
                acc_v, out_hbm.at[pl.ds(row_base + k * C, C)], wsems[b]
            )
        return carry

    lax.fori_loop(0, NCHUNK // 2, _pair, 0)

    for b in range(2):
        k = NCHUNK - 2 + b
        pltpu.make_async_copy(
            accs[b], out_hbm.at[pl.ds(row_base + k * C, C)], wsems[b]
        ).wait()


def _tc_body(x_ref, w_hbm, out_ref, wb, st0, st1, sem0, sem1):
    p = pl.program_id(0)

    @pl.when(p == 0)
    def _():
        stages = (st0, st1)
        sems = (sem0, sem1)
        cp = pltpu.make_async_copy(w_hbm.at[pl.ds(0, VOCAB)], st0, sem0)
        cp.start()
        for c in range(F):
            b = c % 2
            if c + 1 < F:
                nxt = pltpu.make_async_copy(
                    w_hbm.at[pl.ds((c + 1) * VOCAB, VOCAB)], stages[1 - b], sems[1 - b])
                nxt.start()
            pltpu.make_async_copy(
                w_hbm.at[pl.ds(c * VOCAB, VOCAB)], stages[b], sems[b]).wait()
            wb[c, pl.ds(0, VOCAB), :] = stages[b][...].astype(jnp.bfloat16)
        for f in range(F):
            wb[f, pl.ds(VOCAB, VP - VOCAB), :] = jnp.zeros(
                (VP - VOCAB, D), jnp.bfloat16)

    acc = jnp.zeros((PB, D), jnp.float32)
    cols = lax.broadcasted_iota(jnp.int32, (PB, VP), 1)
    for f in range(F):
        idx = x_ref[:, f]
        oh = (cols == idx[:, None]).astype(jnp.bfloat16)
        acc = acc + jnp.dot(oh, wb[f], preferred_element_type=jnp.float32)
    out_ref[...] = acc


_embed_tc = pl.pallas_call(
    _tc_body,
    grid=(PT // PB,),
    in_specs=[
        pl.BlockSpec((PB, F), lambda p: (p, 0)),
        pl.BlockSpec(memory_space=pltpu.HBM),
    ],
    out_specs=pl.BlockSpec((PB, D), lambda p: (p, 0)),
    out_shape=jax.ShapeDtypeStruct((PT, D), jnp.float32),
    scratch_shapes=[
        pltpu.VMEM((F, VP, D), jnp.bfloat16),
        pltpu.VMEM((VOCAB, D), jnp.float32),
        pltpu.VMEM((VOCAB, D), jnp.float32),
        pltpu.SemaphoreType.DMA,
        pltpu.SemaphoreType.DMA,
    ],
)


def kernel(x, W):
    xi = x.astype(jnp.int32).reshape(N, F)
    xf = xi[:NSC].reshape(NW, NCHUNK, G)
    sc_out = _embed_sc(xf, W)
    tc_out = _embed_tc(xi[NSC:], W)
    return jnp.concatenate([sc_out, tc_out], axis=0).reshape(B, S, D)


# speedup vs baseline: 1.1222x; 1.1222x over previous
import functools

import jax
import jax.numpy as jnp
from jax import lax
from jax.experimental import pallas as pl
from jax.experimental.pallas import tpu as pltpu
from jax.experimental.pallas import tpu_sc as plsc

VOCAB = 1000
D = 2048
F = 8
B, S = 2, 2048
N = B * S

NC, NS, L = 2, 16, 16
NW = NC * NS
P_W = N // NW
C = 2
G = C * F
NCHUNK = P_W // C
NB = 3

_mesh = plsc.VectorSubcoreMesh(core_axis_name="c", subcore_axis_name="s")


@functools.partial(
    pl.kernel,
    mesh=_mesh,
    out_type=jax.ShapeDtypeStruct((N, D), jnp.float32),
    scratch_types=[
        pltpu.VMEM((NCHUNK, G), jnp.int32),
        pltpu.VMEM((G, D), jnp.float32),
        pltpu.VMEM((G, D), jnp.float32),
        pltpu.VMEM((G, D), jnp.float32),
        pltpu.VMEM((C, D), jnp.float32),
        pltpu.VMEM((C, D), jnp.float32),
        pltpu.VMEM((C, D), jnp.float32),
        pltpu.SemaphoreType.DMA,
        pltpu.SemaphoreType.DMA,
        pltpu.SemaphoreType.DMA,
        pltpu.SemaphoreType.DMA,
        pltpu.SemaphoreType.DMA,
        pltpu.SemaphoreType.DMA,
    ],
)
def _embed_sc(x_hbm, w_hbm, out_hbm, idx_v, rows0, rows1, rows2,
              acc0, acc1, acc2, sem0, sem1, sem2, wsem0, wsem1, wsem2):
    wid = lax.axis_index("s") * NC + lax.axis_index("c")
    row_base = wid * P_W
    bufs = (rows0, rows1, rows2)
    sems = (sem0, sem1, sem2)
    accs = (acc0, acc1, acc2)
    wsems = (wsem0, wsem1, wsem2)

    pltpu.sync_copy(x_hbm.at[wid], idx_v)
    offs = (lax.iota(jnp.int32, 16) % F) * VOCAB

    def _bias(c, carry):
        idx_v[c, pl.ds(0, L)] = idx_v[c, pl.ds(0, L)] + offs
        return carry

    lax.fori_loop(0, NCHUNK, _bias, 0)

    pltpu.async_copy(w_hbm.at[idx_v.at[0]], rows0, sem0)
    pltpu.async_copy(w_hbm.at[idx_v.at[1]], rows1, sem1)

    def _maybe(cond, fn):
        if isinstance(cond, bool):
            if cond:
                fn()
        else:
            pl.when(cond)(fn)

    def _chunk(k, b):
        nb2 = (b + 2) % NB

        def _prefetch():
            pltpu.async_copy(w_hbm.at[idx_v.at[k + 2]], bufs[nb2], sems[nb2])

        _maybe(k + 2 < NCHUNK, _prefetch)

        pltpu.make_async_copy(w_hbm.at[idx_v.at[k]], bufs[b], sems[b]).wait()
        rows_v = bufs[b]
        acc_v = accs[b]

        def _wb_wait():
            pltpu.make_async_copy(
                acc_v, out_hbm.at[pl.ds(row_base + (k - NB) * C, C)], wsems[b]
            ).wait()

        _maybe(k >= NB, _wb_wait)

        @plsc.parallel_loop(0, D // L, unroll=8)
        def _cols(j):
            col = j * L
            for c in range(C):
                t = [rows_v[c * F + f, pl.ds(col, L)] for f in range(F)]
                while len(t) > 1:
                    t = [a + b2 for a, b2 in zip(t[::2], t[1::2])]
                acc_v[c, pl.ds(col, L)] = t[0]

        pltpu.async_copy(
            acc_v, out_hbm.at[pl.ds(row_base + k * C, C)], wsems[b]
        )

    def _triple(k3, carry):
        for b in range(NB):
            _chunk(k3 * NB + b, b)
        return carry

    lax.fori_loop(0, NCHUNK // NB, _triple, 0)
    _chunk(NCHUNK - 1, (NCHUNK - 1) % NB)

    for k in range(NCHUNK - NB, NCHUNK):
        b = k % NB
        pltpu.make_async_copy(
            accs[b], out_hbm.at[pl.ds(row_base + k * C, C)], wsems[b]
        ).wait()


def kernel(x, W):
    xf = x.astype(jnp.int32).reshape(NW, NCHUNK, G)
    out = _embed_sc(xf, W)
    return out.reshape(B, S, D)
